# baseline (speedup 1.0000x reference)
import functools

import jax
import jax.numpy as jnp
from jax import lax
from jax.experimental import pallas as pl
from jax.experimental.pallas import tpu as pltpu
from jax.experimental.pallas import tpu_sc as plsc

N = 10000
E = 106667
D = 128
D_AUG = 144
ONES_COL = 128
N_PAD = 10240
DUMMY = N
NC = 2
NS = 16
NW = NC * NS
CHUNK = 128
SLAB = N_PAD // NS
RB = 1024
F32 = jnp.float32

NCH1 = (39, 14)
NCH2 = (44, 9)
for _n0, _n1 in (NCH1, NCH2):
    assert NS * (_n0 + _n1) * CHUNK >= E


@functools.lru_cache(maxsize=None)
def _make_sc_segment_sum(d, n0, n1, async0=False):
    mesh = plsc.VectorSubcoreMesh(core_axis_name="c", subcore_axis_name="s",
                                  num_cores=NC, num_subcores=NS)

    def body(table, srci, dsti, out, src_v, dst_v, rows_v, rows2_v, acc,
             sem, sem2):
        c = lax.axis_index("c")
        s = lax.axis_index("s")
        wid = c * NS + s
        base = s * SLAB
        nch = jnp.where(c == 0, n0, n1)

        def zrow(i, _):
            for k in range(d // 16):
                rows_v[i, pl.ds(k * 16, 16)] = jnp.zeros((16,), F32)
            return 0

        def chunk_step(j, _):
            pltpu.async_copy(table.at[src_v.at[j]], rows_v, sem).wait()
            pltpu.sync_copy(rows_v, acc.at[dst_v.at[j]], add=True)
            return 0

        def core0_pipelined(_):
            pltpu.async_copy(table.at[src_v.at[0]], rows_v, sem)

            def step(jj, _):
                j = 2 * jj
                pltpu.async_copy(table.at[src_v.at[j + 1]], rows2_v, sem2)
                pltpu.make_async_copy(table.at[src_v.at[j]], rows_v,
                                      sem).wait()
                pltpu.sync_copy(rows_v, acc.at[dst_v.at[j]], add=True)
                pltpu.async_copy(table.at[src_v.at[j + 2]], rows_v, sem)
                pltpu.make_async_copy(table.at[src_v.at[j + 1]], rows2_v,
                                      sem2).wait()
                pltpu.sync_copy(rows2_v, acc.at[dst_v.at[j + 1]], add=True)
                return 0

            lax.fori_loop(0, (n0 - 2) // 2, step, 0)
            pltpu.async_copy(table.at[src_v.at[n0 - 1]], rows2_v, sem2)
            pltpu.make_async_copy(table.at[src_v.at[n0 - 2]], rows_v,
                                  sem).wait()
            pltpu.sync_copy(rows_v, acc.at[dst_v.at[n0 - 2]], add=True)
            pltpu.make_async_copy(table.at[src_v.at[n0 - 1]], rows2_v,
                                  sem2).wait()
            pltpu.sync_copy(rows2_v, acc.at[dst_v.at[n0 - 1]], add=True)
            return 0

        def core1_sync(_):
            lax.fori_loop(0, n1, chunk_step, 0)
            return 0

        for r in range(3):
            lax.fori_loop(0, CHUNK, zrow, 0)
            for k in range(SLAB // CHUNK):
                pltpu.sync_copy(rows_v, acc.at[pl.ds(base + k * CHUNK, CHUNK)])
            pltpu.sync_copy(srci.at[r, wid], src_v)
            pltpu.sync_copy(dsti.at[r, wid], dst_v)
            plsc.subcore_barrier()

            if async0:
                lax.cond(c == 0, core0_pipelined, core1_sync, 0)
            else:
                lax.fori_loop(0, nch, chunk_step, 0)
            plsc.subcore_barrier()
            pltpu.sync_copy(acc.at[pl.ds(base, SLAB)],
                            out.at[c, r, pl.ds(base, SLAB)])

    scratch = [
        pltpu.VMEM((n0, CHUNK), jnp.int32),
        pltpu.VMEM((n0, CHUNK), jnp.int32),
        pltpu.VMEM((CHUNK, d), F32),
        pltpu.VMEM((CHUNK, d) if async0 else (8, d), F32),
        pltpu.VMEM_SHARED((N_PAD, d), F32),
        pltpu.SemaphoreType.DMA,
        pltpu.SemaphoreType.DMA,
    ]
    return pl.kernel(
        body,
        out_type=jax.ShapeDtypeStruct((NC, 3, N_PAD, d), F32),
        mesh=mesh,
        scratch_types=scratch,
        compiler_params=pltpu.CompilerParams(use_tc_tiling_on_sc=False),
    )


def _tc1_body(p_ref, w_ref, b_ref, h_ref, dinv_ref):
    i = pl.program_id(0)
    p = p_ref[...]
    ssum = p[0] + p[1]
    deg = ssum[:, :, ONES_COL]
    dinv = 1.0 / jnp.maximum(deg, 1.0)
    acc = jnp.zeros((RB, D), F32)
    for r in range(3):
        acc = acc + jnp.dot(ssum[r, :, :D] * dinv[r][:, None], w_ref[r],
                            preferred_element_type=F32)
        acc = acc + b_ref[r][None, :]
    rows = i * RB + lax.broadcasted_iota(jnp.int32, (RB, 1), 0)
    h_ref[...] = jnp.where(rows < N, acc, 0.0)
    dinv_ref[...] = dinv


def _tc2_body(p_ref, dinv_ref, w_ref, b_ref, out_ref):
    p = p_ref[...]
    ssum = p[0] + p[1]
    dinv = dinv_ref[...]
    acc = jnp.zeros((RB, D), F32)
    for r in range(3):
        acc = acc + jnp.dot(ssum[r] * dinv[r][:, None], w_ref[r],
                            preferred_element_type=F32)
        acc = acc + b_ref[r][None, :]
    out_ref[...] = acc


def _tc_combine1(partials, w1s, b1s):
    return pl.pallas_call(
        _tc1_body,
        grid=(N_PAD // RB,),
        in_specs=[
            pl.BlockSpec((NC, 3, RB, D_AUG), lambda i: (0, 0, i, 0)),
            pl.BlockSpec((3, D, D), lambda i: (0, 0, 0)),
            pl.BlockSpec((3, D), lambda i: (0, 0)),
        ],
        out_specs=[
            pl.BlockSpec((RB, D), lambda i: (i, 0)),
            pl.BlockSpec((3, RB), lambda i: (0, i)),
        ],
        out_shape=[
            jax.ShapeDtypeStruct((N_PAD, D), F32),
            jax.ShapeDtypeStruct((3, N_PAD), F32),
        ],
    )(partials, w1s, b1s)


def _tc_combine2(partials, dinv, w2s, b2s):
    return pl.pallas_call(
        _tc2_body,
        grid=(N_PAD // RB,),
        in_specs=[
            pl.BlockSpec((NC, 3, RB, D), lambda i: (0, 0, i, 0)),
            pl.BlockSpec((3, RB), lambda i: (0, i)),
            pl.BlockSpec((3, D, D), lambda i: (0, 0, 0)),
            pl.BlockSpec((3, D), lambda i: (0, 0)),
        ],
        out_specs=pl.BlockSpec((RB, D), lambda i: (i, 0)),
        out_shape=jax.ShapeDtypeStruct((N_PAD, D), F32),
    )(partials, dinv, w2s, b2s)


def _pack_edges(edge_indices, n0, n1):
    cap = NS * (n0 + n1) * CHUNK
    split = NS * n0 * CHUNK
    packed = []
    for part in (0, 1):
        rows = []
        for ei in edge_indices:
            v = jnp.full((cap,), DUMMY, jnp.int32).at[:E].set(
                ei[part].astype(jnp.int32))
            a = v[:split].reshape(NS, n0, CHUNK)
            b = v[split:].reshape(NS, n1, CHUNK)
            b = jnp.pad(b, ((0, 0), (0, n0 - n1), (0, 0)),
                        constant_values=DUMMY)
            rows.append(jnp.concatenate([a, b], axis=0))
        packed.append(jnp.stack(rows))
    return packed[0], packed[1]


def kernel(x, edge_index_r0, edge_index_r1, edge_index_r2,
           W1_r0, b1_r0, W1_r1, b1_r1, W1_r2, b1_r2,
           W2_r0, b2_r0, W2_r1, b2_r1, W2_r2, b2_r2):
    edges = (edge_index_r0, edge_index_r1, edge_index_r2)
    srci1, dsti1 = _pack_edges(edges, *NCH1)
    srci2, dsti2 = _pack_edges(edges, *NCH2)

    xa = jnp.zeros((N_PAD, D_AUG), F32)
    xa = xa.at[:N, :D].set(x.astype(F32))
    xa = xa.at[:N, ONES_COL].set(1.0)

    w1s = jnp.stack([W1_r0, W1_r1, W1_r2])
    b1s = jnp.stack([b1_r0, b1_r1, b1_r2])
    w2s = jnp.stack([W2_r0, W2_r1, W2_r2])
    b2s = jnp.stack([b2_r0, b2_r1, b2_r2])

    p1 = _make_sc_segment_sum(D_AUG, *NCH1)(xa, srci1, dsti1)
    h, dinv = _tc_combine1(p1, w1s, b1s)
    p2 = _make_sc_segment_sum(D, *NCH2, async0=True)(h, srci2, dsti2)
    out = _tc_combine2(p2, dinv, w2s, b2s)
    return out[:N]


# speedup vs baseline: 1.0930x; 1.0930x over previous
import functools

import jax
import jax.numpy as jnp
from jax import lax
from jax.experimental import pallas as pl
from jax.experimental.pallas import tpu as pltpu
from jax.experimental.pallas import tpu_sc as plsc

N = 10000
E = 106667
D = 128
D_AUG = 144
ONES_COL = 128
N_PAD = 10240
DUMMY = N
NC = 2
NS = 16
NW = NC * NS
CHUNK = 128
SLAB = N_PAD // NS
RB = 1024
F32 = jnp.float32

NCH1 = (39, 14)
NCH2 = (48, 5)
for _n0, _n1 in (NCH1, NCH2):
    assert NS * (_n0 + _n1) * CHUNK >= E


@functools.lru_cache(maxsize=None)
def _make_sc_segment_sum(d, n0, n1, async0=False):
    mesh = plsc.VectorSubcoreMesh(core_axis_name="c", subcore_axis_name="s",
                                  num_cores=NC, num_subcores=NS)

    def body(table, srci, dsti, out, src_v, dst_v, rows_v, rows2_v, acc,
             sem, sem2):
        c = lax.axis_index("c")
        s = lax.axis_index("s")
        wid = c * NS + s
        base = s * SLAB
        nch = jnp.where(c == 0, n0, n1)

        def zrow(i, _):
            for k in range(d // 16):
                rows_v[i, pl.ds(k * 16, 16)] = jnp.zeros((16,), F32)
            return 0

        def chunk_step(j, _):
            pltpu.async_copy(table.at[src_v.at[j]], rows_v, sem).wait()
            pltpu.sync_copy(rows_v, acc.at[dst_v.at[j]], add=True)
            return 0

        def core0_pipelined(_):
            pltpu.async_copy(table.at[src_v.at[0]], rows_v, sem)

            def step(jj, _):
                j = 2 * jj
                pltpu.async_copy(table.at[src_v.at[j + 1]], rows2_v, sem2)
                pltpu.make_async_copy(table.at[src_v.at[j]], rows_v,
                                      sem).wait()
                pltpu.sync_copy(rows_v, acc.at[dst_v.at[j]], add=True)
                pltpu.async_copy(table.at[src_v.at[j + 2]], rows_v, sem)
                pltpu.make_async_copy(table.at[src_v.at[j + 1]], rows2_v,
                                      sem2).wait()
                pltpu.sync_copy(rows2_v, acc.at[dst_v.at[j + 1]], add=True)
                return 0

            lax.fori_loop(0, (n0 - 2) // 2, step, 0)
            pltpu.async_copy(table.at[src_v.at[n0 - 1]], rows2_v, sem2)
            pltpu.make_async_copy(table.at[src_v.at[n0 - 2]], rows_v,
                                  sem).wait()
            pltpu.sync_copy(rows_v, acc.at[dst_v.at[n0 - 2]], add=True)
            pltpu.make_async_copy(table.at[src_v.at[n0 - 1]], rows2_v,
                                  sem2).wait()
            pltpu.sync_copy(rows2_v, acc.at[dst_v.at[n0 - 1]], add=True)
            return 0

        def core1_sync(_):
            lax.fori_loop(0, n1, chunk_step, 0)
            return 0

        for r in range(3):
            lax.fori_loop(0, CHUNK, zrow, 0)
            for k in range(SLAB // CHUNK):
                pltpu.sync_copy(rows_v, acc.at[pl.ds(base + k * CHUNK, CHUNK)])
            pltpu.sync_copy(srci.at[r, wid], src_v)
            pltpu.sync_copy(dsti.at[r, wid], dst_v)
            plsc.subcore_barrier()

            if async0:
                lax.cond(c == 0, core0_pipelined, core1_sync, 0)
            else:
                lax.fori_loop(0, nch, chunk_step, 0)
            plsc.subcore_barrier()
            pltpu.sync_copy(acc.at[pl.ds(base, SLAB)],
                            out.at[c, r, pl.ds(base, SLAB)])

    scratch = [
        pltpu.VMEM((n0, CHUNK), jnp.int32),
        pltpu.VMEM((n0, CHUNK), jnp.int32),
        pltpu.VMEM((CHUNK, d), F32),
        pltpu.VMEM((CHUNK, d) if async0 else (8, d), F32),
        pltpu.VMEM_SHARED((N_PAD, d), F32),
        pltpu.SemaphoreType.DMA,
        pltpu.SemaphoreType.DMA,
    ]
    return pl.kernel(
        body,
        out_type=jax.ShapeDtypeStruct((NC, 3, N_PAD, d), F32),
        mesh=mesh,
        scratch_types=scratch,
        compiler_params=pltpu.CompilerParams(use_tc_tiling_on_sc=False),
    )


def _tc1_body(p_ref, w_ref, b_ref, h_ref, dinv_ref):
    i = pl.program_id(0)
    p = p_ref[...]
    ssum = p[0] + p[1]
    deg = ssum[:, :, ONES_COL]
    dinv = 1.0 / jnp.maximum(deg, 1.0)
    acc = jnp.zeros((RB, D), F32)
    for r in range(3):
        acc = acc + jnp.dot(ssum[r, :, :D] * dinv[r][:, None], w_ref[r],
                            preferred_element_type=F32)
        acc = acc + b_ref[r][None, :]
    rows = i * RB + lax.broadcasted_iota(jnp.int32, (RB, 1), 0)
    h_ref[...] = jnp.where(rows < N, acc, 0.0)
    dinv_ref[...] = dinv


def _tc2_body(p_ref, dinv_ref, w_ref, b_ref, out_ref):
    p = p_ref[...]
    ssum = p[0] + p[1]
    dinv = dinv_ref[...]
    acc = jnp.zeros((RB, D), F32)
    for r in range(3):
        acc = acc + jnp.dot(ssum[r] * dinv[r][:, None], w_ref[r],
                            preferred_element_type=F32)
        acc = acc + b_ref[r][None, :]
    out_ref[...] = acc


def _tc_combine1(partials, w1s, b1s):
    return pl.pallas_call(
        _tc1_body,
        grid=(N_PAD // RB,),
        in_specs=[
            pl.BlockSpec((NC, 3, RB, D_AUG), lambda i: (0, 0, i, 0)),
            pl.BlockSpec((3, D, D), lambda i: (0, 0, 0)),
            pl.BlockSpec((3, D), lambda i: (0, 0)),
        ],
        out_specs=[
            pl.BlockSpec((RB, D), lambda i: (i, 0)),
            pl.BlockSpec((3, RB), lambda i: (0, i)),
        ],
        out_shape=[
            jax.ShapeDtypeStruct((N_PAD, D), F32),
            jax.ShapeDtypeStruct((3, N_PAD), F32),
        ],
    )(partials, w1s, b1s)


def _tc_combine2(partials, dinv, w2s, b2s):
    return pl.pallas_call(
        _tc2_body,
        grid=(N_PAD // RB,),
        in_specs=[
            pl.BlockSpec((NC, 3, RB, D), lambda i: (0, 0, i, 0)),
            pl.BlockSpec((3, RB), lambda i: (0, i)),
            pl.BlockSpec((3, D, D), lambda i: (0, 0, 0)),
            pl.BlockSpec((3, D), lambda i: (0, 0)),
        ],
        out_specs=pl.BlockSpec((RB, D), lambda i: (i, 0)),
        out_shape=jax.ShapeDtypeStruct((N_PAD, D), F32),
    )(partials, dinv, w2s, b2s)


def _pack_edges(edge_indices, n0, n1):
    cap = NS * (n0 + n1) * CHUNK
    split = NS * n0 * CHUNK
    packed = []
    for part in (0, 1):
        rows = []
        for ei in edge_indices:
            v = jnp.full((cap,), DUMMY, jnp.int32).at[:E].set(
                ei[part].astype(jnp.int32))
            a = v[:split].reshape(NS, n0, CHUNK)
            b = v[split:].reshape(NS, n1, CHUNK)
            b = jnp.pad(b, ((0, 0), (0, n0 - n1), (0, 0)),
                        constant_values=DUMMY)
            rows.append(jnp.concatenate([a, b], axis=0))
        packed.append(jnp.stack(rows))
    return packed[0], packed[1]


def kernel(x, edge_index_r0, edge_index_r1, edge_index_r2,
           W1_r0, b1_r0, W1_r1, b1_r1, W1_r2, b1_r2,
           W2_r0, b2_r0, W2_r1, b2_r1, W2_r2, b2_r2):
    edges = (edge_index_r0, edge_index_r1, edge_index_r2)
    srci1, dsti1 = _pack_edges(edges, *NCH1)
    srci2, dsti2 = _pack_edges(edges, *NCH2)

    xa = jnp.zeros((N_PAD, D_AUG), F32)
    xa = xa.at[:N, :D].set(x.astype(F32))
    xa = xa.at[:N, ONES_COL].set(1.0)

    w1s = jnp.stack([W1_r0, W1_r1, W1_r2])
    b1s = jnp.stack([b1_r0, b1_r1, b1_r2])
    w2s = jnp.stack([W2_r0, W2_r1, W2_r2])
    b2s = jnp.stack([b2_r0, b2_r1, b2_r2])

    p1 = _make_sc_segment_sum(D_AUG, *NCH1)(xa, srci1, dsti1)
    h, dinv = _tc_combine1(p1, w1s, b1s)
    p2 = _make_sc_segment_sum(D, *NCH2, async0=True)(h, srci2, dsti2)
    out = _tc_combine2(p2, dinv, w2s, b2s)
    return out[:N]
